# baseline (speedup 1.0000x reference)
import functools

import jax
import jax.numpy as jnp
from jax import lax
from jax.experimental import pallas as pl
from jax.experimental.pallas import tpu as pltpu
from jax.experimental.pallas import tpu_sc as plsc

N = 50000
K = 16
E = N * K
TN = 400
RT = TN * K
G = N // TN

_NC = 2
_NS = 16
_NW = _NC * _NS
_CH = 1000

_HI = jax.lax.Precision.HIGHEST
_LO = jax.lax.Precision.DEFAULT


def _sc_gather(table, idx):
    e = idx.shape[0]
    d = table.shape[1]
    bpw = e // _NW
    nch = bpw // _CH
    mesh = plsc.VectorSubcoreMesh(core_axis_name="c", subcore_axis_name="s")

    @functools.partial(
        pl.kernel,
        mesh=mesh,
        out_type=jax.ShapeDtypeStruct((e, d), jnp.float32),
        compiler_params=pltpu.CompilerParams(use_tc_tiling_on_sc=False),
        scratch_types=[
            pltpu.VMEM((_CH,), jnp.int32),
            pltpu.VMEM((_CH, d), jnp.float32),
            pltpu.SemaphoreType.DMA,
        ],
    )
    def k(table_hbm, idx_hbm, out_hbm, idx_v, rows_v, sem):
        wid = lax.axis_index("s") * _NC + lax.axis_index("c")
        base = wid * bpw

        def body(c, carry):
            off = base + c * _CH
            pltpu.sync_copy(idx_hbm.at[pl.ds(off, _CH)], idx_v)
            pltpu.async_copy(table_hbm.at[idx_v], rows_v, sem).wait()
            pltpu.sync_copy(rows_v, out_hbm.at[pl.ds(off, _CH)])
            return carry

        lax.fori_loop(0, nch, body, 0)

    return k(table, idx)


def _stats_accum(ref, vals):
    s = jnp.sum(vals, axis=0, keepdims=True)
    q = jnp.sum(vals * vals, axis=0, keepdims=True)
    part = jnp.broadcast_to(jnp.concatenate([s, q], axis=1), ref.shape)
    i = pl.program_id(0)

    @pl.when(i == 0)
    def _():
        ref[...] = part

    @pl.when(i > 0)
    def _():
        ref[...] = ref[...] + part


def _fold_affine(stats, g, be, cnt):
    c = g.shape[1]
    s, q = stats[0:1, :c], stats[0:1, c:2 * c]
    m = s / cnt
    v = q / cnt - m * m
    sc = g / jnp.sqrt(v + 1e-5)
    return sc, be - m * sc


def _copy_in(hbm_ref, vmem_ref, sem, rows):
    i = pl.program_id(0)
    cp = pltpu.make_async_copy(hbm_ref.at[pl.ds(i * rows, rows), :],
                               vmem_ref, sem)
    cp.start()
    cp.wait()


def _tc1_body(gath_hbm, xyz_ref, wa_ref, wb_ref, wd_ref, b1_ref, y1_ref,
              stats_ref, gath_v, sem):
    _copy_in(gath_hbm, gath_v, sem, RT)
    xj = gath_v[...][:, 0:3]
    xi = jnp.broadcast_to(xyz_ref[...][:, None, :], (TN, K, 3)).reshape(RT, 3)
    rel = xi - xj
    dis = jnp.sqrt(jnp.sum(rel * rel, axis=1, keepdims=True))
    pa = jnp.dot(xyz_ref[...], wa_ref[...], precision=_LO)
    pab = jnp.broadcast_to(pa[:, None, :], (TN, K, 16)).reshape(RT, 16)
    y1 = (pab + jnp.dot(xj, wb_ref[...], precision=_LO)
          + dis * wd_ref[0:1, :] + b1_ref[0:1, :])
    y1_ref[...] = y1
    _stats_accum(stats_ref, y1)


def _softmax_pool(f_cat, att):
    a3 = att.reshape(TN, K, att.shape[1])
    f3 = f_cat.reshape(TN, K, f_cat.shape[1])
    m = jnp.max(a3, axis=1, keepdims=True)
    ex = jnp.exp(a3 - m)
    sm = jnp.sum(ex, axis=1, keepdims=True)
    return jnp.sum(f3 * (ex / sm), axis=1)


def _tc2_body(y1_ref, gath_hbm, stats1_ref, bn1_ref, awc_ref, abc_ref,
              mw1t_ref, mb1_ref, ym1_ref, y2_ref, stats_m1_ref, stats_y2_ref,
              gath_v, sem):
    _copy_in(gath_hbm, gath_v, sem, RT)
    sc1, sh1 = _fold_affine(stats1_ref[...], bn1_ref[0:1, :], bn1_ref[1:2, :], E)
    f_xyz = jnp.maximum(y1_ref[...] * sc1 + sh1, 0.0)
    f_cat = jnp.concatenate([gath_v[...][:, 16:32], f_xyz], axis=1)
    av = jnp.dot(f_cat, awc_ref[...], precision=_LO) + abc_ref[0:1, :]
    att = av[:, 0:32]
    y2 = av[:, 32:48]
    agg = _softmax_pool(f_cat, att)
    y_m1 = jnp.dot(agg, mw1t_ref[...], precision=_HI) + mb1_ref[0:1, :]
    ym1_ref[...] = y_m1
    y2_ref[...] = y2
    _stats_accum(stats_m1_ref, y_m1)
    _stats_accum(stats_y2_ref, y2)


def _tc3_body(y2_ref, gath2_hbm, stats_y2_ref, bn2_ref, stats_m1_ref,
              bnm1_ref, aw2t_ref, ab2_ref, mw2t_ref, mb2_ref,
              ym2_ref, stats_m2_ref, gath2_v, sem):
    _copy_in(gath2_hbm, gath2_v, sem, RT)
    sc2, sh2 = _fold_affine(stats_y2_ref[...], bn2_ref[0:1, :], bn2_ref[1:2, :], E)
    scm, shm = _fold_affine(stats_m1_ref[...], bnm1_ref[0:1, :], bnm1_ref[1:2, :], N)
    f_xyz2 = jnp.maximum(y2_ref[...] * sc2 + sh2, 0.0)
    f_nb2 = jnp.maximum(gath2_v[...] * scm + shm, 0.0)
    f_cat2 = jnp.concatenate([f_nb2, f_xyz2], axis=1)
    att2 = jnp.dot(f_cat2, aw2t_ref[...], precision=_LO) + ab2_ref[0:1, :]
    agg2 = _softmax_pool(f_cat2, att2)
    y_m2 = jnp.dot(agg2, mw2t_ref[...], precision=_HI) + mb2_ref[0:1, :]
    ym2_ref[...] = y_m2
    _stats_accum(stats_m2_ref, y_m2)


def _tc4_body(ym2_ref, stats_m2_ref, bnm2_ref, out_ref):
    scm2, shm2 = _fold_affine(
        stats_m2_ref[...], bnm2_ref[0:1, :], bnm2_ref[1:2, :], N)
    y = jnp.maximum(ym2_ref[...] * scm2 + shm2, 0.0)
    out_ref[...] = y.T


def _full(shape):
    return pl.BlockSpec(shape, lambda i: (0, 0))


def kernel(xyz, feature, neigh_idx, W1, b1, g1, be1, aW1, ab1, mW1, mb1,
           mg1, mbe1, W2, b2, g2, be2, aW2, ab2, mW2, mb2, mg2, mbe2):
    f32 = jnp.float32
    P = xyz[0].astype(f32)
    feat = feature[0, :, :, 0].T.astype(f32)
    table1 = jnp.concatenate(
        [P, jnp.zeros((N, 13), f32), feat], axis=1)
    idx = neigh_idx.reshape(E).astype(jnp.int32)

    gath1 = _sc_gather(table1, idx)

    W1t = W1.T
    wA = W1t[1:4, :] + W1t[4:7, :]
    wB = W1t[7:10, :] - W1t[1:4, :]
    wD = jnp.broadcast_to(W1t[0:1, :], (8, 16))

    y1, stats1 = pl.pallas_call(
        _tc1_body,
        grid=(G,),
        in_specs=[
            pl.BlockSpec(memory_space=pl.ANY),
            pl.BlockSpec((TN, 3), lambda i: (i, 0)),
            _full((3, 16)),
            _full((3, 16)),
            _full((8, 16)),
            _full((8, 16)),
        ],
        out_specs=[
            pl.BlockSpec((RT, 16), lambda i: (i, 0)),
            _full((8, 32)),
        ],
        out_shape=[
            jax.ShapeDtypeStruct((E, 16), f32),
            jax.ShapeDtypeStruct((8, 32), f32),
        ],
        scratch_shapes=[pltpu.VMEM((RT, 32), f32), pltpu.SemaphoreType.DMA],
    )(gath1, P, wA, wB, wD, jnp.broadcast_to(b1, (8, 16)))

    aWc = jnp.concatenate(
        [aW1.T, jnp.concatenate([jnp.zeros((16, 16), f32), W2.T], axis=0)],
        axis=1)
    abc = jnp.concatenate([ab1, b2])
    bn1 = jnp.stack([g1, be1])

    y_m1, y2, stats_m1, stats_y2 = pl.pallas_call(
        _tc2_body,
        grid=(G,),
        in_specs=[
            pl.BlockSpec((RT, 16), lambda i: (i, 0)),
            pl.BlockSpec(memory_space=pl.ANY),
            _full((8, 32)),
            _full((2, 16)),
            _full((32, 48)),
            _full((8, 48)),
            _full((32, 16)),
            _full((8, 16)),
        ],
        out_specs=[
            pl.BlockSpec((TN, 16), lambda i: (i, 0)),
            pl.BlockSpec((RT, 16), lambda i: (i, 0)),
            _full((8, 32)),
            _full((8, 32)),
        ],
        out_shape=[
            jax.ShapeDtypeStruct((N, 16), f32),
            jax.ShapeDtypeStruct((E, 16), f32),
            jax.ShapeDtypeStruct((8, 32), f32),
            jax.ShapeDtypeStruct((8, 32), f32),
        ],
        scratch_shapes=[pltpu.VMEM((RT, 32), f32), pltpu.SemaphoreType.DMA],
    )(y1, gath1, stats1, bn1, aWc, jnp.broadcast_to(abc, (8, 48)),
      mW1.T, jnp.broadcast_to(mb1, (8, 16)))

    gath2 = _sc_gather(y_m1, idx)

    bn2 = jnp.stack([g2, be2])
    bnm1 = jnp.stack([mg1, mbe1])

    y_m2, stats_m2 = pl.pallas_call(
        _tc3_body,
        grid=(G,),
        in_specs=[
            pl.BlockSpec((RT, 16), lambda i: (i, 0)),
            pl.BlockSpec(memory_space=pl.ANY),
            _full((8, 32)),
            _full((2, 16)),
            _full((8, 32)),
            _full((2, 16)),
            _full((32, 32)),
            _full((8, 32)),
            _full((32, 32)),
            _full((8, 32)),
        ],
        out_specs=[
            pl.BlockSpec((TN, 32), lambda i: (i, 0)),
            _full((8, 64)),
        ],
        out_shape=[
            jax.ShapeDtypeStruct((N, 32), f32),
            jax.ShapeDtypeStruct((8, 64), f32),
        ],
        scratch_shapes=[pltpu.VMEM((RT, 16), f32), pltpu.SemaphoreType.DMA],
    )(y2, gath2, stats_y2, bn2, stats_m1, bnm1, aW2.T,
      jnp.broadcast_to(ab2, (8, 32)), mW2.T, jnp.broadcast_to(mb2, (8, 32)))

    bnm2 = jnp.stack([mg2, mbe2])

    out = pl.pallas_call(
        _tc4_body,
        grid=(1,),
        in_specs=[
            _full((N, 32)),
            _full((8, 64)),
            _full((2, 32)),
        ],
        out_specs=_full((32, N)),
        out_shape=jax.ShapeDtypeStruct((32, N), f32),
    )(y_m2, stats_m2, bnm2)

    return out.reshape(1, 32, N, 1)


# speedup vs baseline: 1.5386x; 1.5386x over previous
import functools

import jax
import jax.numpy as jnp
from jax import lax
from jax.experimental import pallas as pl
from jax.experimental.pallas import tpu as pltpu
from jax.experimental.pallas import tpu_sc as plsc

N = 50000
K = 16
E = N * K
TN = 400
RT = TN * K
G = N // TN

_NC = 2
_NS = 16
_NW = _NC * _NS
_CH = 1000

_HI = jax.lax.Precision.HIGHEST
_LO = jax.lax.Precision.DEFAULT


def _sc_gather(table, idx):
    e = idx.shape[0]
    d = table.shape[1]
    bpw = e // _NW
    nch = bpw // _CH
    mesh = plsc.VectorSubcoreMesh(core_axis_name="c", subcore_axis_name="s")

    @functools.partial(
        pl.kernel,
        mesh=mesh,
        out_type=jax.ShapeDtypeStruct((e, d), jnp.float32),
        compiler_params=pltpu.CompilerParams(use_tc_tiling_on_sc=False),
        scratch_types=[
            pltpu.VMEM((_CH,), jnp.int32),
            pltpu.VMEM((_CH, d), jnp.float32),
            pltpu.SemaphoreType.DMA,
        ],
    )
    def k(table_hbm, idx_hbm, out_hbm, idx_v, rows_v, sem):
        wid = lax.axis_index("s") * _NC + lax.axis_index("c")
        base = wid * bpw

        def body(c, carry):
            off = base + c * _CH
            pltpu.sync_copy(idx_hbm.at[pl.ds(off, _CH)], idx_v)
            pltpu.async_copy(table_hbm.at[idx_v], rows_v, sem).wait()
            pltpu.sync_copy(rows_v, out_hbm.at[pl.ds(off, _CH)])
            return carry

        lax.fori_loop(0, nch, body, 0)

    return k(table, idx)


def _stats_accum(ref, vals):
    s = jnp.sum(vals, axis=0, keepdims=True)
    q = jnp.sum(vals * vals, axis=0, keepdims=True)
    part = jnp.broadcast_to(jnp.concatenate([s, q], axis=1), ref.shape)
    i = pl.program_id(0)

    @pl.when(i == 0)
    def _():
        ref[...] = part

    @pl.when(i > 0)
    def _():
        ref[...] = ref[...] + part


def _fold_affine(stats, g, be, cnt):
    c = g.shape[1]
    s, q = stats[0:1, :c], stats[0:1, c:2 * c]
    m = s / cnt
    v = q / cnt - m * m
    sc = g / jnp.sqrt(v + 1e-5)
    return sc, be - m * sc


def _copy_in(hbm_ref, buf, sems, rows):
    i = pl.program_id(0)
    n = pl.num_programs(0)
    slot = lax.rem(i, 2)
    nxt = lax.rem(i + 1, 2)

    @pl.when(i == 0)
    def _():
        pltpu.make_async_copy(hbm_ref.at[pl.ds(0, rows), :],
                              buf.at[0], sems.at[0]).start()

    @pl.when(i + 1 < n)
    def _():
        pltpu.make_async_copy(hbm_ref.at[pl.ds((i + 1) * rows, rows), :],
                              buf.at[nxt], sems.at[nxt]).start()

    pltpu.make_async_copy(hbm_ref.at[pl.ds(i * rows, rows), :],
                          buf.at[slot], sems.at[slot]).wait()
    return buf[slot]


def _tc1_body(gath_hbm, xyz_ref, wa_ref, wb_ref, wd_ref, b1_ref, y1_ref,
              stats_ref, gath_v, sem):
    gv = _copy_in(gath_hbm, gath_v, sem, RT)
    xj = gv[:, 0:3]
    xi = jnp.broadcast_to(xyz_ref[...][:, None, :], (TN, K, 3)).reshape(RT, 3)
    rel = xi - xj
    dis = jnp.sqrt(jnp.sum(rel * rel, axis=1, keepdims=True))
    pa = jnp.dot(xyz_ref[...], wa_ref[...], precision=_LO)
    pab = jnp.broadcast_to(pa[:, None, :], (TN, K, 16)).reshape(RT, 16)
    y1 = (pab + jnp.dot(xj, wb_ref[...], precision=_LO)
          + dis * wd_ref[0:1, :] + b1_ref[0:1, :])
    y1_ref[...] = y1
    _stats_accum(stats_ref, y1)


def _softmax_pool(f_cat, att):
    a3 = att.reshape(TN, K, att.shape[1])
    f3 = f_cat.reshape(TN, K, f_cat.shape[1])
    m = jnp.max(a3, axis=1, keepdims=True)
    ex = jnp.exp(a3 - m)
    sm = jnp.sum(ex, axis=1, keepdims=True)
    return jnp.sum(f3 * (ex / sm), axis=1)


def _tc2_body(y1_ref, gath_hbm, stats1_ref, bn1_ref, awc_ref, abc_ref,
              mw1t_ref, mb1_ref, ym1_ref, y2_ref, stats_m1_ref, stats_y2_ref,
              gath_v, sem):
    gv = _copy_in(gath_hbm, gath_v, sem, RT)
    sc1, sh1 = _fold_affine(stats1_ref[...], bn1_ref[0:1, :], bn1_ref[1:2, :], E)
    f_xyz = jnp.maximum(y1_ref[...] * sc1 + sh1, 0.0)
    f_cat = jnp.concatenate([gv[:, 16:32], f_xyz], axis=1)
    av = jnp.dot(f_cat, awc_ref[...], precision=_LO) + abc_ref[0:1, :]
    att = av[:, 0:32]
    y2 = av[:, 32:48]
    agg = _softmax_pool(f_cat, att)
    y_m1 = jnp.dot(agg, mw1t_ref[...], precision=_HI) + mb1_ref[0:1, :]
    ym1_ref[...] = y_m1
    y2_ref[...] = y2
    _stats_accum(stats_m1_ref, y_m1)
    _stats_accum(stats_y2_ref, y2)


def _tc3_body(y2_ref, gath2_hbm, stats_y2_ref, bn2_ref, stats_m1_ref,
              bnm1_ref, aw2t_ref, ab2_ref, mw2t_ref, mb2_ref,
              ym2_ref, stats_m2_ref, gath2_v, sem):
    gv2 = _copy_in(gath2_hbm, gath2_v, sem, RT)
    sc2, sh2 = _fold_affine(stats_y2_ref[...], bn2_ref[0:1, :], bn2_ref[1:2, :], E)
    scm, shm = _fold_affine(stats_m1_ref[...], bnm1_ref[0:1, :], bnm1_ref[1:2, :], N)
    f_xyz2 = jnp.maximum(y2_ref[...] * sc2 + sh2, 0.0)
    f_nb2 = jnp.maximum(gv2 * scm + shm, 0.0)
    f_cat2 = jnp.concatenate([f_nb2, f_xyz2], axis=1)
    att2 = jnp.dot(f_cat2, aw2t_ref[...], precision=_LO) + ab2_ref[0:1, :]
    agg2 = _softmax_pool(f_cat2, att2)
    y_m2 = jnp.dot(agg2, mw2t_ref[...], precision=_HI) + mb2_ref[0:1, :]
    ym2_ref[...] = y_m2
    _stats_accum(stats_m2_ref, y_m2)


def _tc4_body(ym2_ref, stats_m2_ref, bnm2_ref, out_ref):
    scm2, shm2 = _fold_affine(
        stats_m2_ref[...], bnm2_ref[0:1, :], bnm2_ref[1:2, :], N)
    y = jnp.maximum(ym2_ref[...] * scm2 + shm2, 0.0)
    out_ref[...] = y.T


def _full(shape):
    return pl.BlockSpec(shape, lambda i: (0, 0))


def kernel(xyz, feature, neigh_idx, W1, b1, g1, be1, aW1, ab1, mW1, mb1,
           mg1, mbe1, W2, b2, g2, be2, aW2, ab2, mW2, mb2, mg2, mbe2):
    f32 = jnp.float32
    P = xyz[0].astype(f32)
    feat = feature[0, :, :, 0].T.astype(f32)
    table1 = jnp.concatenate(
        [P, jnp.zeros((N, 13), f32), feat], axis=1)
    idx = neigh_idx.reshape(E).astype(jnp.int32)

    gath1 = _sc_gather(table1, idx)

    W1t = W1.T
    wA = W1t[1:4, :] + W1t[4:7, :]
    wB = W1t[7:10, :] - W1t[1:4, :]
    wD = jnp.broadcast_to(W1t[0:1, :], (8, 16))

    y1, stats1 = pl.pallas_call(
        _tc1_body,
        grid=(G,),
        in_specs=[
            pl.BlockSpec(memory_space=pl.ANY),
            pl.BlockSpec((TN, 3), lambda i: (i, 0)),
            _full((3, 16)),
            _full((3, 16)),
            _full((8, 16)),
            _full((8, 16)),
        ],
        out_specs=[
            pl.BlockSpec((RT, 16), lambda i: (i, 0)),
            _full((8, 32)),
        ],
        out_shape=[
            jax.ShapeDtypeStruct((E, 16), f32),
            jax.ShapeDtypeStruct((8, 32), f32),
        ],
        scratch_shapes=[pltpu.VMEM((2, RT, 32), f32),
                        pltpu.SemaphoreType.DMA((2,))],
    )(gath1, P, wA, wB, wD, jnp.broadcast_to(b1, (8, 16)))

    aWc = jnp.concatenate(
        [aW1.T, jnp.concatenate([jnp.zeros((16, 16), f32), W2.T], axis=0)],
        axis=1)
    abc = jnp.concatenate([ab1, b2])
    bn1 = jnp.stack([g1, be1])

    y_m1, y2, stats_m1, stats_y2 = pl.pallas_call(
        _tc2_body,
        grid=(G,),
        in_specs=[
            pl.BlockSpec((RT, 16), lambda i: (i, 0)),
            pl.BlockSpec(memory_space=pl.ANY),
            _full((8, 32)),
            _full((2, 16)),
            _full((32, 48)),
            _full((8, 48)),
            _full((32, 16)),
            _full((8, 16)),
        ],
        out_specs=[
            pl.BlockSpec((TN, 16), lambda i: (i, 0)),
            pl.BlockSpec((RT, 16), lambda i: (i, 0)),
            _full((8, 32)),
            _full((8, 32)),
        ],
        out_shape=[
            jax.ShapeDtypeStruct((N, 16), f32),
            jax.ShapeDtypeStruct((E, 16), f32),
            jax.ShapeDtypeStruct((8, 32), f32),
            jax.ShapeDtypeStruct((8, 32), f32),
        ],
        scratch_shapes=[pltpu.VMEM((2, RT, 32), f32),
                        pltpu.SemaphoreType.DMA((2,))],
    )(y1, gath1, stats1, bn1, aWc, jnp.broadcast_to(abc, (8, 48)),
      mW1.T, jnp.broadcast_to(mb1, (8, 16)))

    gath2 = _sc_gather(y_m1, idx)

    bn2 = jnp.stack([g2, be2])
    bnm1 = jnp.stack([mg1, mbe1])

    y_m2, stats_m2 = pl.pallas_call(
        _tc3_body,
        grid=(G,),
        in_specs=[
            pl.BlockSpec((RT, 16), lambda i: (i, 0)),
            pl.BlockSpec(memory_space=pl.ANY),
            _full((8, 32)),
            _full((2, 16)),
            _full((8, 32)),
            _full((2, 16)),
            _full((32, 32)),
            _full((8, 32)),
            _full((32, 32)),
            _full((8, 32)),
        ],
        out_specs=[
            pl.BlockSpec((TN, 32), lambda i: (i, 0)),
            _full((8, 64)),
        ],
        out_shape=[
            jax.ShapeDtypeStruct((N, 32), f32),
            jax.ShapeDtypeStruct((8, 64), f32),
        ],
        scratch_shapes=[pltpu.VMEM((2, RT, 16), f32),
                        pltpu.SemaphoreType.DMA((2,))],
    )(y2, gath2, stats_y2, bn2, stats_m1, bnm1, aW2.T,
      jnp.broadcast_to(ab2, (8, 32)), mW2.T, jnp.broadcast_to(mb2, (8, 32)))

    bnm2 = jnp.stack([mg2, mbe2])

    out = pl.pallas_call(
        _tc4_body,
        grid=(1,),
        in_specs=[
            _full((N, 32)),
            _full((8, 64)),
            _full((2, 32)),
        ],
        out_specs=_full((32, N)),
        out_shape=jax.ShapeDtypeStruct((32, N), f32),
    )(y_m2, stats_m2, bnm2)

    return out.reshape(1, 32, N, 1)


# speedup vs baseline: 1.5762x; 1.0245x over previous
import functools

import jax
import jax.numpy as jnp
from jax import lax
from jax.experimental import pallas as pl
from jax.experimental.pallas import tpu as pltpu
from jax.experimental.pallas import tpu_sc as plsc

N = 50000
K = 16
E = N * K
TN = 400
RT = TN * K
G = N // TN

_NC = 2
_NS = 16
_NW = _NC * _NS
_CH = 1000

_HI = jax.lax.Precision.HIGHEST
_LO = jax.lax.Precision.DEFAULT


def _sc_gather(table, idx):
    e = idx.shape[0]
    d = table.shape[1]
    bpw = e // _NW
    nch = bpw // _CH
    mesh = plsc.VectorSubcoreMesh(core_axis_name="c", subcore_axis_name="s")

    @functools.partial(
        pl.kernel,
        mesh=mesh,
        out_type=jax.ShapeDtypeStruct((e, d), jnp.float32),
        compiler_params=pltpu.CompilerParams(use_tc_tiling_on_sc=False),
        scratch_types=[
            pltpu.VMEM((_CH,), jnp.int32),
            pltpu.VMEM((_CH, d), jnp.float32),
            pltpu.SemaphoreType.DMA,
        ],
    )
    def k(table_hbm, idx_hbm, out_hbm, idx_v, rows_v, sem):
        wid = lax.axis_index("s") * _NC + lax.axis_index("c")
        base = wid * bpw

        def body(c, carry):
            off = base + c * _CH
            pltpu.sync_copy(idx_hbm.at[pl.ds(off, _CH)], idx_v)
            pltpu.async_copy(table_hbm.at[idx_v], rows_v, sem).wait()
            pltpu.sync_copy(rows_v, out_hbm.at[pl.ds(off, _CH)])
            return carry

        lax.fori_loop(0, nch, body, 0)

    return k(table, idx)


def _stats_accum(ref, vals):
    s = jnp.sum(vals, axis=0, keepdims=True)
    q = jnp.sum(vals * vals, axis=0, keepdims=True)
    part = jnp.broadcast_to(jnp.concatenate([s, q], axis=1), ref.shape)
    i = pl.program_id(0)

    @pl.when(i == 0)
    def _():
        ref[...] = part

    @pl.when(i > 0)
    def _():
        ref[...] = ref[...] + part


def _fold_affine(stats, g, be, cnt):
    c = g.shape[1]
    s, q = stats[0:1, :c], stats[0:1, c:2 * c]
    m = s / cnt
    v = q / cnt - m * m
    sc = g / jnp.sqrt(v + 1e-5)
    return sc, be - m * sc


def _tc1_body(gath_ref, xyz_ref, wa_ref, wb_ref, wd_ref, b1_ref, y1_ref,
              stats_ref):
    xj = gath_ref[...][:, 0:3]
    xi = jnp.broadcast_to(xyz_ref[...][:, None, :], (TN, K, 3)).reshape(RT, 3)
    rel = xi - xj
    dis = jnp.sqrt(jnp.sum(rel * rel, axis=1, keepdims=True))
    pa = jnp.dot(xyz_ref[...], wa_ref[...], precision=_LO)
    pab = jnp.broadcast_to(pa[:, None, :], (TN, K, 16)).reshape(RT, 16)
    y1 = (pab + jnp.dot(xj, wb_ref[...], precision=_LO)
          + dis * wd_ref[0:1, :] + b1_ref[0:1, :])
    y1_ref[...] = y1
    _stats_accum(stats_ref, y1)


def _softmax_pool(f_cat, att):
    a3 = att.reshape(TN, K, att.shape[1])
    f3 = f_cat.reshape(TN, K, f_cat.shape[1])
    m = jnp.max(a3, axis=1, keepdims=True)
    ex = jnp.exp(a3 - m)
    sm = jnp.sum(ex, axis=1, keepdims=True)
    return jnp.sum(f3 * (ex / sm), axis=1)


def _tc2_body(y1_ref, gath_ref, stats1_ref, bn1_ref, awc_ref, abc_ref,
              mw1t_ref, mb1_ref, ym1_ref, y2_ref, stats_m1_ref, stats_y2_ref):
    sc1, sh1 = _fold_affine(stats1_ref[...], bn1_ref[0:1, :], bn1_ref[1:2, :], E)
    f_xyz = jnp.maximum(y1_ref[...] * sc1 + sh1, 0.0)
    f_cat = jnp.concatenate([gath_ref[...][:, 16:32], f_xyz], axis=1)
    av = jnp.dot(f_cat, awc_ref[...], precision=_LO) + abc_ref[0:1, :]
    att = av[:, 0:32]
    y2 = av[:, 32:48]
    agg = _softmax_pool(f_cat, att)
    y_m1 = jnp.dot(agg, mw1t_ref[...], precision=_LO) + mb1_ref[0:1, :]
    ym1_ref[...] = y_m1
    y2_ref[...] = y2
    _stats_accum(stats_m1_ref, y_m1)
    _stats_accum(stats_y2_ref, y2)


def _tc3_body(y2_ref, gath2_ref, stats_y2_ref, bn2_ref, stats_m1_ref,
              bnm1_ref, aw2t_ref, ab2_ref, mw2t_ref, mb2_ref,
              ym2_ref, stats_m2_ref):
    sc2, sh2 = _fold_affine(stats_y2_ref[...], bn2_ref[0:1, :], bn2_ref[1:2, :], E)
    scm, shm = _fold_affine(stats_m1_ref[...], bnm1_ref[0:1, :], bnm1_ref[1:2, :], N)
    f_xyz2 = jnp.maximum(y2_ref[...] * sc2 + sh2, 0.0)
    f_nb2 = jnp.maximum(gath2_ref[...] * scm + shm, 0.0)
    f_cat2 = jnp.concatenate([f_nb2, f_xyz2], axis=1)
    att2 = jnp.dot(f_cat2, aw2t_ref[...], precision=_LO) + ab2_ref[0:1, :]
    agg2 = _softmax_pool(f_cat2, att2)
    y_m2 = jnp.dot(agg2, mw2t_ref[...], precision=_LO) + mb2_ref[0:1, :]
    ym2_ref[...] = y_m2
    _stats_accum(stats_m2_ref, y_m2)


def _tc4_body(ym2_ref, stats_m2_ref, bnm2_ref, out_ref):
    scm2, shm2 = _fold_affine(
        stats_m2_ref[...], bnm2_ref[0:1, :], bnm2_ref[1:2, :], N)
    y = jnp.maximum(ym2_ref[...] * scm2 + shm2, 0.0)
    out_ref[...] = y.T


def _full(shape):
    return pl.BlockSpec(shape, lambda i: (0, 0))


def kernel(xyz, feature, neigh_idx, W1, b1, g1, be1, aW1, ab1, mW1, mb1,
           mg1, mbe1, W2, b2, g2, be2, aW2, ab2, mW2, mb2, mg2, mbe2):
    f32 = jnp.float32
    P = xyz[0].astype(f32)
    feat = feature[0, :, :, 0].T.astype(f32)
    table1 = jnp.concatenate(
        [P, jnp.zeros((N, 13), f32), feat], axis=1)
    idx = neigh_idx.reshape(E).astype(jnp.int32)

    gath1 = _sc_gather(table1, idx)

    W1t = W1.T
    wA = W1t[1:4, :] + W1t[4:7, :]
    wB = W1t[7:10, :] - W1t[1:4, :]
    wD = jnp.broadcast_to(W1t[0:1, :], (8, 16))

    y1, stats1 = pl.pallas_call(
        _tc1_body,
        grid=(G,),
        in_specs=[
            pl.BlockSpec((RT, 32), lambda i: (i, 0)),
            pl.BlockSpec((TN, 3), lambda i: (i, 0)),
            _full((3, 16)),
            _full((3, 16)),
            _full((8, 16)),
            _full((8, 16)),
        ],
        out_specs=[
            pl.BlockSpec((RT, 16), lambda i: (i, 0)),
            _full((8, 32)),
        ],
        out_shape=[
            jax.ShapeDtypeStruct((E, 16), f32),
            jax.ShapeDtypeStruct((8, 32), f32),
        ],
    )(gath1, P, wA, wB, wD, jnp.broadcast_to(b1, (8, 16)))

    aWc = jnp.concatenate(
        [aW1.T, jnp.concatenate([jnp.zeros((16, 16), f32), W2.T], axis=0)],
        axis=1)
    abc = jnp.concatenate([ab1, b2])
    bn1 = jnp.stack([g1, be1])

    y_m1, y2, stats_m1, stats_y2 = pl.pallas_call(
        _tc2_body,
        grid=(G,),
        in_specs=[
            pl.BlockSpec((RT, 16), lambda i: (i, 0)),
            pl.BlockSpec((RT, 32), lambda i: (i, 0)),
            _full((8, 32)),
            _full((2, 16)),
            _full((32, 48)),
            _full((8, 48)),
            _full((32, 16)),
            _full((8, 16)),
        ],
        out_specs=[
            pl.BlockSpec((TN, 16), lambda i: (i, 0)),
            pl.BlockSpec((RT, 16), lambda i: (i, 0)),
            _full((8, 32)),
            _full((8, 32)),
        ],
        out_shape=[
            jax.ShapeDtypeStruct((N, 16), f32),
            jax.ShapeDtypeStruct((E, 16), f32),
            jax.ShapeDtypeStruct((8, 32), f32),
            jax.ShapeDtypeStruct((8, 32), f32),
        ],
    )(y1, gath1, stats1, bn1, aWc, jnp.broadcast_to(abc, (8, 48)),
      mW1.T, jnp.broadcast_to(mb1, (8, 16)))

    gath2 = _sc_gather(y_m1, idx)

    bn2 = jnp.stack([g2, be2])
    bnm1 = jnp.stack([mg1, mbe1])

    y_m2, stats_m2 = pl.pallas_call(
        _tc3_body,
        grid=(G,),
        in_specs=[
            pl.BlockSpec((RT, 16), lambda i: (i, 0)),
            pl.BlockSpec((RT, 16), lambda i: (i, 0)),
            _full((8, 32)),
            _full((2, 16)),
            _full((8, 32)),
            _full((2, 16)),
            _full((32, 32)),
            _full((8, 32)),
            _full((32, 32)),
            _full((8, 32)),
        ],
        out_specs=[
            pl.BlockSpec((TN, 32), lambda i: (i, 0)),
            _full((8, 64)),
        ],
        out_shape=[
            jax.ShapeDtypeStruct((N, 32), f32),
            jax.ShapeDtypeStruct((8, 64), f32),
        ],
    )(y2, gath2, stats_y2, bn2, stats_m1, bnm1, aW2.T,
      jnp.broadcast_to(ab2, (8, 32)), mW2.T, jnp.broadcast_to(mb2, (8, 32)))

    bnm2 = jnp.stack([mg2, mbe2])

    out = pl.pallas_call(
        _tc4_body,
        grid=(1,),
        in_specs=[
            _full((N, 32)),
            _full((8, 64)),
            _full((2, 32)),
        ],
        out_specs=_full((32, N)),
        out_shape=jax.ShapeDtypeStruct((32, N), f32),
    )(y_m2, stats_m2, bnm2)

    return out.reshape(1, 32, N, 1)
